# baseline (speedup 1.0000x reference)
import jax
import jax.numpy as jnp
from jax import lax
from jax.experimental import pallas as pl
from jax.experimental.pallas import tpu as pltpu
from jax.experimental.pallas import tpu_sc as plsc

C1 = 768
C2 = 768
H = 224
W = 224
HW = H * W
K = 25088

NS = 16
PER_W = K // NS
CHUNK = 112
NCH = PER_W // CHUNK
ZCH = HW // NS

TILE = 1792
NSTEP = HW // TILE


def _mask_sc_body(idx_hbm, mask_hbm, zeros_v, idx_v, ones_v, sem):
    wid = lax.axis_index("s")

    def _zfill(i, carry):
        zeros_v[pl.ds(i * 16, 16)] = jnp.zeros((16,), jnp.float32)
        return carry

    lax.fori_loop(0, ZCH // 16, _zfill, 0)
    pltpu.sync_copy(zeros_v, mask_hbm.at[pl.ds(wid * ZCH, ZCH)])
    plsc.subcore_barrier()

    pltpu.sync_copy(idx_hbm.at[wid], idx_v)
    for i in range(0, CHUNK, 16):
        ones_v[pl.ds(i, 16)] = jnp.ones((16,), jnp.float32)
    copies = [
        pltpu.async_copy(ones_v, mask_hbm.at[idx_v.at[j]], sem)
        for j in range(NCH)
    ]
    for cp in copies:
        cp.wait()


_mask_sc = pl.kernel(
    _mask_sc_body,
    out_type=jax.ShapeDtypeStruct((HW,), jnp.float32),
    mesh=plsc.VectorSubcoreMesh(
        core_axis_name="c", subcore_axis_name="s", num_cores=1
    ),
    scratch_types=[
        pltpu.VMEM((ZCH,), jnp.float32),
        pltpu.VMEM((NCH, CHUNK), jnp.int32),
        pltpu.VMEM((CHUNK,), jnp.float32),
        pltpu.SemaphoreType.DMA,
    ],
)


def _mm_body(w_ref, x_ref, m_ref, o_ref):
    o_ref[...] = lax.dot(
        w_ref[...], x_ref[...] * m_ref[...],
        preferred_element_type=jnp.float32,
    )


_masked_mm = pl.pallas_call(
    _mm_body,
    grid=(NSTEP,),
    in_specs=[
        pl.BlockSpec((C2, C1), lambda j: (0, 0)),
        pl.BlockSpec((C1, TILE), lambda j: (0, j)),
        pl.BlockSpec((1, TILE), lambda j: (0, j)),
    ],
    out_specs=pl.BlockSpec((C2, TILE), lambda j: (0, j)),
    out_shape=jax.ShapeDtypeStruct((C2, HW), jnp.float32),
)


def kernel(c1hw, indices, weight):
    in2 = c1hw.reshape(C1, HW)
    idx3 = indices.astype(jnp.int32).reshape(NS, NCH, CHUNK)
    mask = _mask_sc(idx3)
    out2 = _masked_mm(weight, in2, mask.reshape(1, HW))
    return out2.reshape(1, C2, H, W)
